# baseline (speedup 1.0000x reference)
import functools

import jax
import jax.numpy as jnp
from jax import lax
from jax.experimental import pallas as pl

HIDDEN = 4096
EXPERTS = 64
K = 8
BLOCK_TOKENS = 1024
CHUNKS = 1


def _topk8_t(logits_t):
    b = logits_t.shape[1]
    iota = lax.broadcasted_iota(jnp.int32, (EXPERTS, b), 0).astype(jnp.float32)
    neg_inf = jnp.float32(-jnp.inf)

    vals = logits_t
    top_v = []
    top_i = []
    for _ in range(K):
        m = jnp.max(vals, axis=0, keepdims=True)
        idx = jnp.min(jnp.where(vals == m, iota, jnp.float32(EXPERTS)),
                      axis=0, keepdims=True)
        top_v.append(m)
        top_i.append(idx)
        vals = jnp.where(iota == idx, neg_inf, vals)

    tv = jnp.concatenate(top_v, axis=0)
    ti = jnp.concatenate(top_i, axis=0)
    ew = jnp.exp(tv - tv[:1])
    w = ew / jnp.sum(ew, axis=0, keepdims=True)
    return w.T, ti.T.astype(jnp.int32)


def _router_block(x_ref, w_ref, logits_ref, weights_ref, idx_ref):
    w = w_ref[...]
    c = BLOCK_TOKENS // CHUNKS
    for i in range(CHUNKS):
        rows = pl.ds(i * c, c)
        x = x_ref[rows, :]
        logits_t = lax.dot_general(w, x, (((0,), (1,)), ((), ())),
                                   preferred_element_type=jnp.float32)
        logits_ref[rows, :] = logits_t.T
        wts, idx = _topk8_t(logits_t)
        weights_ref[rows, :] = wts
        idx_ref[rows, :] = idx


@functools.partial(jax.jit, static_argnames=())
def _router(x2d, W):
    n = x2d.shape[0]
    grid = (n // BLOCK_TOKENS,)
    return pl.pallas_call(
        _router_block,
        grid=grid,
        in_specs=[
            pl.BlockSpec((BLOCK_TOKENS, HIDDEN), lambda i: (i, 0)),
            pl.BlockSpec((HIDDEN, EXPERTS), lambda i: (0, 0)),
        ],
        out_specs=[
            pl.BlockSpec((BLOCK_TOKENS, EXPERTS), lambda i: (i, 0)),
            pl.BlockSpec((BLOCK_TOKENS, K), lambda i: (i, 0)),
            pl.BlockSpec((BLOCK_TOKENS, K), lambda i: (i, 0)),
        ],
        out_shape=[
            jax.ShapeDtypeStruct((n, EXPERTS), jnp.float32),
            jax.ShapeDtypeStruct((n, K), jnp.float32),
            jax.ShapeDtypeStruct((n, K), jnp.int32),
        ],
    )(x2d, W)


def kernel(hidden_states, W):
    batch, seq, hidden = hidden_states.shape
    x2d = hidden_states.reshape(batch * seq, hidden)
    logits, weights, idx = _router(x2d, W)
    return (
        weights.reshape(batch, seq, K),
        idx.reshape(batch, seq, K),
        logits.reshape(batch, seq, EXPERTS),
    )
